# baseline (speedup 1.0000x reference)
import functools

import jax
import jax.numpy as jnp
from jax import lax
from jax.experimental import pallas as pl
from jax.experimental.pallas import tpu as pltpu
from jax.experimental.pallas import tpu_sc as plsc


def _sc_geometry():
    try:
        info = plsc.get_sparse_core_info()
        return info.num_cores, info.num_subcores
    except Exception:
        return 2, 16


def _make_repack(S, G, D, SA, GA, NC, NS):
    NW = NC * NS
    mesh = plsc.VectorSubcoreMesh(
        core_axis_name="c", subcore_axis_name="s", num_cores=NC, num_subcores=NS
    )
    sf, gf = SA // 128, GA // 128
    GB = (S * D // 128 + 7) // 8 * 8
    ni_f = (sf + NW - 1) // NW
    ni_g = (gf + NW - 1) // NW

    @functools.partial(
        pl.kernel,
        mesh=mesh,
        out_type=jax.ShapeDtypeStruct((GB + G * D // 128, 128), jnp.float32),
        compiler_params=pltpu.CompilerParams(needs_layout_passes=False),
        scratch_types=[
            pltpu.VMEM((D, 128), jnp.float32),
            pltpu.VMEM((128 * D // 128, 128), jnp.float32),
            pltpu.SemaphoreType.DMA,
        ],
    )
    def k(fw_t, gw_t, out2d, av, pk, sem):
        wid = lax.axis_index("s") * NC + lax.axis_index("c")
        iota = lax.iota(jnp.int32, 16)

        def do_chunk(src, r0, base_row):
            r0 = pl.multiple_of(r0, 128)
            pltpu.async_copy(src.at[:, pl.ds(r0, 128)], av, sem).wait()
            for c in range(D):
                for rr in range(128 // 16):
                    vals = av[c, pl.ds(rr * 16, 16)]
                    flat = (rr * 16 + iota) * D + c
                    plsc.store_scatter(
                        pk, [lax.shift_right_logical(flat, 7), flat & 127], vals
                    )
            pltpu.sync_copy(
                pk,
                out2d.at[
                    pl.ds(
                        pl.multiple_of(base_row + r0 * D // 128, 8),
                        128 * D // 128,
                    )
                ],
            )

        def fixed_body(i, _):
            cid = wid + i * NW

            @pl.when(cid < sf)
            def _():
                do_chunk(fw_t, cid * 128, 0)

            return 0

        lax.fori_loop(0, ni_f, fixed_body, 0)

        def grad_body(i, _):
            cid = wid + i * NW

            @pl.when(cid < gf)
            def _():
                do_chunk(gw_t, cid * 128, GB)

            return 0

        lax.fori_loop(0, ni_g, grad_body, 0)

    return k


def _make_lookup(B, H, D, S, G, SA, GA, NC, NS):
    NW = NC * NS
    nbw = (B // 128) // NW
    mesh = plsc.VectorSubcoreMesh(
        core_axis_name="c", subcore_axis_name="s", num_cores=NC, num_subcores=NS
    )
    PW = 4 * D
    GB = (S * D // 128 + 7) // 8 * 8

    @functools.partial(
        pl.kernel,
        mesh=mesh,
        out_type=jax.ShapeDtypeStruct((H, D, B), jnp.float32),
        compiler_params=pltpu.CompilerParams(needs_layout_passes=False),
        scratch_types=[
            pltpu.VMEM((H, nbw * 128), jnp.int32),
            pltpu.VMEM((128,), jnp.int32),
            pltpu.VMEM((144,), jnp.int32),
            pltpu.VMEM((128, PW), jnp.float32),
            pltpu.VMEM((D, 128), jnp.float32),
            pltpu.VMEM((160,), jnp.int32),
            pltpu.VMEM((160,), jnp.int32),
            pltpu.VMEM((160,), jnp.int32),
            pltpu.VMEM((16, PW), jnp.float32),
            pltpu.SemaphoreType.DMA,
        ],
    )
    def k(x_t, rmtab, tail_tab, out_t, xw, i4, sw, gbuf, och, pos, tsl, ti4,
          tgbuf, sem):
        wid = lax.axis_index("s") * NC + lax.axis_index("c")
        iota = lax.iota(jnp.int32, 16)
        bw0 = pl.multiple_of(wid * (nbw * 128), 128)
        pltpu.sync_copy(x_t.at[:, pl.ds(bw0, nbw * 128)], xw)

        def chunk_body(ci, _):
            h = ci // nbw
            bq = ci % nbw

            def pre(kk, cnt):
                iv = xw[h, pl.ds(bq * 128 + kk * 16, 16)]
                m_f = (iv >= SA) & (iv < S)
                m_g = iv >= (S + GA)
                m = m_f | m_g
                slot = jnp.where(iv < S, iv - SA, iv - (S + GA) + (S - SA))
                sr = jnp.where(
                    iv < S,
                    lax.shift_right_logical(iv, 2),
                    GB + lax.shift_right_logical(iv - S, 2),
                )
                i4[pl.ds(kk * 16, 16)] = jnp.where(m, 0, sr)
                sw[pl.ds(kk * 16, 16)] = jnp.where(m, 0, (iv & 3) * D)
                csum = plsc.cumsum(jnp.where(m, 1, 0))
                tgt = jnp.where(m, cnt + csum - 1, 136 + iota)
                plsc.store_scatter(pos, [tgt], kk * 16 + iota)
                plsc.store_scatter(tsl, [tgt], slot)
                plsc.store_scatter(ti4, [tgt], lax.shift_right_logical(slot, 2))
                return cnt + csum[15]

            cnt = lax.fori_loop(0, 8, pre, jnp.int32(0))
            ti4[pl.ds(cnt, 16)] = jnp.zeros((16,), jnp.int32)

            pltpu.async_copy(rmtab.at[i4], gbuf, sem).wait()

            def row_body(j, _):
                soff = sw[pl.ds(j, 16)][0]
                jv = jnp.zeros((16,), jnp.int32) + j
                for e0 in range(0, D, 16):
                    vals = gbuf[j, pl.ds(soff + e0, 16)]
                    plsc.store_scatter(och, [e0 + iota, jv], vals)
                return 0

            lax.fori_loop(0, 128, row_body, 0)

            nfix = (cnt + 15) // 16

            def fix_blk(f, _):
                pltpu.async_copy(
                    tail_tab.at[ti4.at[pl.ds(f * 16, 16)]], tgbuf, sem
                ).wait()

                def fix_lane(l, _):
                    i = f * 16 + l
                    j = pos[pl.ds(i, 16)][0]
                    slot = tsl[pl.ds(i, 16)][0]
                    soff = (slot & 3) * D
                    jv = jnp.zeros((16,), jnp.int32) + j
                    for e0 in range(0, D, 16):
                        vals = tgbuf[l, pl.ds(soff + e0, 16)]
                        plsc.store_scatter(och, [e0 + iota, jv], vals)
                    return 0

                lax.fori_loop(0, jnp.minimum(16, cnt - f * 16), fix_lane, 0)
                return 0

            lax.fori_loop(0, nfix, fix_blk, 0)

            pltpu.sync_copy(
                och,
                out_t.at[h, :, pl.ds(pl.multiple_of(bw0 + bq * 128, 128), 128)],
            )
            return 0

        lax.fori_loop(0, H * nbw, chunk_body, 0)

    return k


def kernel(x, fixed_weight, grad_weight):
    S, D = fixed_weight.shape
    G = grad_weight.shape[0]
    Bb, H = x.shape
    NC, NS = _sc_geometry()
    SA, GA = (S // 128) * 128, (G // 128) * 128

    fw_t = fixed_weight.T
    gw_t = grad_weight.T
    x_t = x.T.astype(jnp.int32)

    tail_rows = jnp.concatenate([fixed_weight[SA:], grad_weight[GA:]], axis=0)
    npad = (-tail_rows.shape[0]) % 4
    tail_rows = jnp.pad(tail_rows, ((0, npad), (0, 0)))
    tail_tab = tail_rows.reshape(-1, 4 * D)

    rmtab = _make_repack(S, G, D, SA, GA, NC, NS)(fw_t, gw_t)

    out_t = _make_lookup(Bb, H, D, S, G, SA, GA, NC, NS)(x_t, rmtab, tail_tab)
    return out_t.transpose(2, 0, 1)


# speedup vs baseline: 1.1416x; 1.1416x over previous
import functools

import jax
import jax.numpy as jnp
from jax import lax
from jax.experimental import pallas as pl
from jax.experimental.pallas import tpu as pltpu
from jax.experimental.pallas import tpu_sc as plsc


def _sc_geometry():
    try:
        info = plsc.get_sparse_core_info()
        return info.num_cores, info.num_subcores
    except Exception:
        return 2, 16


def _make_repack(S, G, D, SA, GA, NC, NS):
    NW = NC * NS
    mesh = plsc.VectorSubcoreMesh(
        core_axis_name="c", subcore_axis_name="s", num_cores=NC, num_subcores=NS
    )
    sf, gf = SA // 128, GA // 128
    GB = (S * D // 128 + 7) // 8 * 8
    ni_f = (sf + NW - 1) // NW
    ni_g = (gf + NW - 1) // NW

    @functools.partial(
        pl.kernel,
        mesh=mesh,
        out_type=jax.ShapeDtypeStruct((GB + G * D // 128, 128), jnp.float32),
        compiler_params=pltpu.CompilerParams(needs_layout_passes=False),
        scratch_types=[
            pltpu.VMEM((D, 128), jnp.float32),
            pltpu.VMEM((128 * D // 128, 128), jnp.float32),
            pltpu.SemaphoreType.DMA,
        ],
    )
    def k(fw_t, gw_t, out2d, av, pk, sem):
        wid = lax.axis_index("s") * NC + lax.axis_index("c")
        iota = lax.iota(jnp.int32, 16)

        def do_chunk(src, r0, base_row):
            r0 = pl.multiple_of(r0, 128)
            pltpu.async_copy(src.at[:, pl.ds(r0, 128)], av, sem).wait()
            for c in range(D):
                for rr in range(128 // 16):
                    vals = av[c, pl.ds(rr * 16, 16)]
                    flat = (rr * 16 + iota) * D + c
                    plsc.store_scatter(
                        pk, [lax.shift_right_logical(flat, 7), flat & 127], vals
                    )
            pltpu.sync_copy(
                pk,
                out2d.at[
                    pl.ds(
                        pl.multiple_of(base_row + r0 * D // 128, 8),
                        128 * D // 128,
                    )
                ],
            )

        def fixed_body(i, _):
            cid = wid + i * NW

            @pl.when(cid < sf)
            def _():
                do_chunk(fw_t, cid * 128, 0)

            return 0

        lax.fori_loop(0, ni_f, fixed_body, 0)

        def grad_body(i, _):
            cid = wid + i * NW

            @pl.when(cid < gf)
            def _():
                do_chunk(gw_t, cid * 128, GB)

            return 0

        lax.fori_loop(0, ni_g, grad_body, 0)

    return k


def _make_lookup(B, H, D, S, G, SA, GA, NC, NS):
    NW = NC * NS
    nbw = (B // 128) // NW
    mesh = plsc.VectorSubcoreMesh(
        core_axis_name="c", subcore_axis_name="s", num_cores=NC, num_subcores=NS
    )
    PW = 4 * D
    GB = (S * D // 128 + 7) // 8 * 8

    @functools.partial(
        pl.kernel,
        mesh=mesh,
        out_type=jax.ShapeDtypeStruct((H, D, B), jnp.float32),
        compiler_params=pltpu.CompilerParams(needs_layout_passes=False),
        scratch_types=[
            pltpu.VMEM((H, nbw * 128), jnp.int32),
            pltpu.VMEM((2, 128), jnp.int32),
            pltpu.VMEM((2, 144), jnp.int32),
            pltpu.VMEM((2, 16), jnp.int32),
            pltpu.VMEM((2 * 176,), jnp.int32),
            pltpu.VMEM((2 * 176,), jnp.int32),
            pltpu.VMEM((2 * 176,), jnp.int32),
            pltpu.VMEM((128, PW), jnp.float32),
            pltpu.VMEM((128, PW), jnp.float32),
            pltpu.VMEM((D, 128), jnp.float32),
            pltpu.VMEM((16, PW), jnp.float32),
            pltpu.SemaphoreType.DMA,
            pltpu.SemaphoreType.DMA,
            pltpu.SemaphoreType.DMA,
        ],
    )
    def k(x_t, rmtab, tail_tab, out_t, xw, i4, sw, cv, pos, tsl, ti4,
          g0, g1, och, tgbuf, gsem0, gsem1, tsem):
        wid = lax.axis_index("s") * NC + lax.axis_index("c")
        iota = lax.iota(jnp.int32, 16)
        bw0 = pl.multiple_of(wid * (nbw * 128), 128)
        pltpu.sync_copy(x_t.at[:, pl.ds(bw0, nbw * 128)], xw)
        gbufs = (g0, g1)
        gsems = (gsem0, gsem1)
        nchw = H * nbw

        def pre_issue(c, b):
            h = c // nbw
            bq = c % nbw
            off = b * 176

            def pre(kk, cnt):
                iv = xw[h, pl.ds(bq * 128 + kk * 16, 16)]
                m = ((iv >= SA) & (iv < S)) | (iv >= (S + GA))
                slot = jnp.where(iv < S, iv - SA, iv - (S + GA) + (S - SA))
                sr = jnp.where(
                    iv < S,
                    lax.shift_right_logical(iv, 2),
                    GB + lax.shift_right_logical(iv - S, 2),
                )
                i4[b, pl.ds(kk * 16, 16)] = jnp.where(m, 0, sr)
                sw[b, pl.ds(kk * 16, 16)] = jnp.where(m, 0, (iv & 3) * D)
                csum = plsc.cumsum(jnp.where(m, 1, 0))
                tgt = jnp.where(m, off + cnt + csum - 1, off + 144 + iota)
                plsc.store_scatter(pos, [tgt], kk * 16 + iota)
                plsc.store_scatter(tsl, [tgt], slot)
                plsc.store_scatter(ti4, [tgt], lax.shift_right_logical(slot, 2))
                return cnt + csum[15]

            cnt = lax.fori_loop(0, 8, pre, jnp.int32(0))
            ti4[pl.ds(off + cnt, 16)] = jnp.zeros((16,), jnp.int32)
            cv[b, pl.ds(0, 16)] = jnp.zeros((16,), jnp.int32) + cnt
            pltpu.async_copy(rmtab.at[i4.at[b]], gbufs[b], gsems[b])

        def consume(c, b):
            h = c // nbw
            bq = c % nbw
            off = b * 176
            gbuf = gbufs[b]
            pltpu.make_async_copy(rmtab.at[i4.at[b]], gbuf, gsems[b]).wait()
            cnt = cv[b, pl.ds(0, 16)][0]

            def row_body(j2, _):
                for jj in range(2):
                    j = j2 * 2 + jj
                    soff = sw[b, pl.ds(j, 16)][0]
                    jv = jnp.zeros((16,), jnp.int32) + j
                    for e0 in range(0, D, 16):
                        vals = gbuf[j, pl.ds(soff + e0, 16)]
                        plsc.store_scatter(och, [e0 + iota, jv], vals)
                return 0

            lax.fori_loop(0, 64, row_body, 0)

            nfix = (cnt + 15) // 16

            def fix_blk(f, _):
                pltpu.async_copy(
                    tail_tab.at[ti4.at[pl.ds(off + f * 16, 16)]], tgbuf, tsem
                ).wait()

                def fix_lane(l, _):
                    i = off + f * 16 + l
                    j = pos[pl.ds(i, 16)][0]
                    slot = tsl[pl.ds(i, 16)][0]
                    soff = (slot & 3) * D
                    jv = jnp.zeros((16,), jnp.int32) + j
                    for e0 in range(0, D, 16):
                        vals = tgbuf[l, pl.ds(soff + e0, 16)]
                        plsc.store_scatter(och, [e0 + iota, jv], vals)
                    return 0

                lax.fori_loop(0, jnp.minimum(16, cnt - f * 16), fix_lane, 0)
                return 0

            lax.fori_loop(0, nfix, fix_blk, 0)

            pltpu.sync_copy(
                och,
                out_t.at[h, :, pl.ds(pl.multiple_of(bw0 + bq * 128, 128), 128)],
            )

        pre_issue(jnp.int32(0), 0)

        def step(s2, _):
            for bb in range(2):
                c = s2 * 2 + bb

                @pl.when(c + 1 < nchw)
                def _():
                    pre_issue(c + 1, bb ^ 1)

                consume(c, bb)
            return 0

        lax.fori_loop(0, nchw // 2, step, 0)

    return k


def kernel(x, fixed_weight, grad_weight):
    S, D = fixed_weight.shape
    G = grad_weight.shape[0]
    Bb, H = x.shape
    NC, NS = _sc_geometry()
    SA, GA = (S // 128) * 128, (G // 128) * 128

    fw_t = fixed_weight.T
    gw_t = grad_weight.T
    x_t = x.T.astype(jnp.int32)

    tail_rows = jnp.concatenate([fixed_weight[SA:], grad_weight[GA:]], axis=0)
    npad = (-tail_rows.shape[0]) % 4
    tail_rows = jnp.pad(tail_rows, ((0, npad), (0, 0)))
    tail_tab = tail_rows.reshape(-1, 4 * D)

    rmtab = _make_repack(S, G, D, SA, GA, NC, NS)(fw_t, gw_t)

    out_t = _make_lookup(Bb, H, D, S, G, SA, GA, NC, NS)(x_t, rmtab, tail_tab)
    return out_t.transpose(2, 0, 1)


# speedup vs baseline: 1.2843x; 1.1250x over previous
import functools

import jax
import jax.numpy as jnp
from jax import lax
from jax.experimental import pallas as pl
from jax.experimental.pallas import tpu as pltpu
from jax.experimental.pallas import tpu_sc as plsc


def _sc_geometry():
    try:
        info = plsc.get_sparse_core_info()
        return info.num_cores, info.num_subcores
    except Exception:
        return 2, 16


def _make_repack(S, G, D, SA, GA, NC, NS):
    NW = NC * NS
    mesh = plsc.VectorSubcoreMesh(
        core_axis_name="c", subcore_axis_name="s", num_cores=NC, num_subcores=NS
    )
    sf, gf = SA // 128, GA // 128
    GB = (S * D // 128 + 7) // 8 * 8
    ni_f = (sf + NW - 1) // NW
    ni_g = (gf + NW - 1) // NW

    @functools.partial(
        pl.kernel,
        mesh=mesh,
        out_type=jax.ShapeDtypeStruct((GB + G * D // 128, 128), jnp.float32),
        compiler_params=pltpu.CompilerParams(needs_layout_passes=False),
        scratch_types=[
            pltpu.VMEM((D, 128), jnp.float32),
            pltpu.VMEM((D, 128), jnp.float32),
            pltpu.VMEM((128 * D // 128, 128), jnp.float32),
            pltpu.SemaphoreType.DMA,
            pltpu.SemaphoreType.DMA,
        ],
    )
    def k(fw_t, gw_t, out2d, av0, av1, pk, rsem0, rsem1):
        wid = lax.axis_index("s") * NC + lax.axis_index("c")
        iota = lax.iota(jnp.int32, 16)
        avs = (av0, av1)
        rsems = (rsem0, rsem1)

        def fire_read(src, r0, bb):
            r0 = pl.multiple_of(r0, 128)
            pltpu.async_copy(src.at[:, pl.ds(r0, 128)], avs[bb], rsems[bb])

        def wait_read(src, r0, bb):
            r0 = pl.multiple_of(r0, 128)
            pltpu.make_async_copy(
                src.at[:, pl.ds(r0, 128)], avs[bb], rsems[bb]
            ).wait()

        def pack_write(bb, r0, base_row):
            av = avs[bb]
            for c in range(D):
                for rr in range(128 // 16):
                    vals = av[c, pl.ds(rr * 16, 16)]
                    flat = (rr * 16 + iota) * D + c
                    plsc.store_scatter(
                        pk, [lax.shift_right_logical(flat, 7), flat & 127], vals
                    )
            pltpu.sync_copy(
                pk,
                out2d.at[
                    pl.ds(
                        pl.multiple_of(base_row + r0 * D // 128, 8),
                        128 * D // 128,
                    )
                ],
            )

        @pl.when(wid < sf)
        def _():
            fire_read(fw_t, wid * 128, 0)

        def fixed_step(s2, _):
            for bb in range(2):
                i = s2 * 2 + bb
                cid = wid + i * NW

                @pl.when(cid + NW < sf)
                def _():
                    fire_read(fw_t, (cid + NW) * 128, bb ^ 1)

                @pl.when(cid < sf)
                def _():
                    wait_read(fw_t, cid * 128, bb)
                    pack_write(bb, cid * 128, 0)

            return 0

        lax.fori_loop(0, (ni_f + 1) // 2, fixed_step, 0)

        def grad_body(i, _):
            cid = wid + i * NW

            @pl.when(cid < gf)
            def _():
                fire_read(gw_t, cid * 128, 0)
                wait_read(gw_t, cid * 128, 0)
                pack_write(0, cid * 128, GB)

            return 0

        lax.fori_loop(0, ni_g, grad_body, 0)

    return k


def _make_lookup(B, H, D, S, G, SA, GA, NC, NS):
    NW = NC * NS
    nbw = (B // 128) // NW
    mesh = plsc.VectorSubcoreMesh(
        core_axis_name="c", subcore_axis_name="s", num_cores=NC, num_subcores=NS
    )
    PW = 4 * D
    GB = (S * D // 128 + 7) // 8 * 8

    @functools.partial(
        pl.kernel,
        mesh=mesh,
        out_type=jax.ShapeDtypeStruct((H, D, B), jnp.float32),
        compiler_params=pltpu.CompilerParams(needs_layout_passes=False),
        scratch_types=[
            pltpu.VMEM((H, nbw * 128), jnp.int32),
            pltpu.VMEM((2, 128), jnp.int32),
            pltpu.VMEM((2, 144), jnp.int32),
            pltpu.VMEM((2, 16), jnp.int32),
            pltpu.VMEM((2 * 176,), jnp.int32),
            pltpu.VMEM((2 * 176,), jnp.int32),
            pltpu.VMEM((2 * 176,), jnp.int32),
            pltpu.VMEM((128, PW), jnp.float32),
            pltpu.VMEM((128, PW), jnp.float32),
            pltpu.VMEM((D, 128), jnp.float32),
            pltpu.VMEM((16, PW), jnp.float32),
            pltpu.SemaphoreType.DMA,
            pltpu.SemaphoreType.DMA,
            pltpu.SemaphoreType.DMA,
        ],
    )
    def k(x_t, rmtab, tail_tab, out_t, xw, i4, sw, cv, pos, tsl, ti4,
          g0, g1, och, tgbuf, gsem0, gsem1, tsem):
        wid = lax.axis_index("s") * NC + lax.axis_index("c")
        iota = lax.iota(jnp.int32, 16)
        bw0 = pl.multiple_of(wid * (nbw * 128), 128)
        pltpu.sync_copy(x_t.at[:, pl.ds(bw0, nbw * 128)], xw)
        gbufs = (g0, g1)
        gsems = (gsem0, gsem1)
        nchw = H * nbw

        def pre_issue(c, b):
            h = c // nbw
            bq = c % nbw
            off = b * 176

            def pre(kk, cnt):
                iv = xw[h, pl.ds(bq * 128 + kk * 16, 16)]
                m = ((iv >= SA) & (iv < S)) | (iv >= (S + GA))
                slot = jnp.where(iv < S, iv - SA, iv - (S + GA) + (S - SA))
                sr = jnp.where(
                    iv < S,
                    lax.shift_right_logical(iv, 2),
                    GB + lax.shift_right_logical(iv - S, 2),
                )
                i4[b, pl.ds(kk * 16, 16)] = jnp.where(m, 0, sr)
                sw[b, pl.ds(kk * 16, 16)] = jnp.where(m, 0, (iv & 3) * D)
                csum = plsc.cumsum(jnp.where(m, 1, 0))
                tgt = jnp.where(m, off + cnt + csum - 1, off + 144 + iota)
                plsc.store_scatter(pos, [tgt], kk * 16 + iota)
                plsc.store_scatter(tsl, [tgt], slot)
                plsc.store_scatter(ti4, [tgt], lax.shift_right_logical(slot, 2))
                return cnt + csum[15]

            cnt = lax.fori_loop(0, 8, pre, jnp.int32(0))
            ti4[pl.ds(off + cnt, 16)] = jnp.zeros((16,), jnp.int32)
            cv[b, pl.ds(0, 16)] = jnp.zeros((16,), jnp.int32) + cnt
            pltpu.async_copy(rmtab.at[i4.at[b]], gbufs[b], gsems[b])

        def consume(c, b):
            h = c // nbw
            bq = c % nbw
            off = b * 176
            gbuf = gbufs[b]
            pltpu.make_async_copy(rmtab.at[i4.at[b]], gbuf, gsems[b]).wait()
            cnt = cv[b, pl.ds(0, 16)][0]

            def row_body(j2, _):
                for jj in range(2):
                    j = j2 * 2 + jj
                    soff = sw[b, pl.ds(j, 16)][0]
                    jv = jnp.zeros((16,), jnp.int32) + j
                    for e0 in range(0, D, 16):
                        vals = gbuf[j, pl.ds(soff + e0, 16)]
                        plsc.store_scatter(och, [e0 + iota, jv], vals)
                return 0

            lax.fori_loop(0, 64, row_body, 0)

            nfix = (cnt + 15) // 16

            def fix_blk(f, _):
                pltpu.async_copy(
                    tail_tab.at[ti4.at[pl.ds(off + f * 16, 16)]], tgbuf, tsem
                ).wait()

                def fix_lane(l, _):
                    i = off + f * 16 + l
                    j = pos[pl.ds(i, 16)][0]
                    slot = tsl[pl.ds(i, 16)][0]
                    soff = (slot & 3) * D
                    jv = jnp.zeros((16,), jnp.int32) + j
                    for e0 in range(0, D, 16):
                        vals = tgbuf[l, pl.ds(soff + e0, 16)]
                        plsc.store_scatter(och, [e0 + iota, jv], vals)
                    return 0

                lax.fori_loop(0, jnp.minimum(16, cnt - f * 16), fix_lane, 0)
                return 0

            lax.fori_loop(0, nfix, fix_blk, 0)

            pltpu.sync_copy(
                och,
                out_t.at[h, :, pl.ds(pl.multiple_of(bw0 + bq * 128, 128), 128)],
            )

        pre_issue(jnp.int32(0), 0)

        def step(s2, _):
            for bb in range(2):
                c = s2 * 2 + bb

                @pl.when(c + 1 < nchw)
                def _():
                    pre_issue(c + 1, bb ^ 1)

                consume(c, bb)
            return 0

        lax.fori_loop(0, nchw // 2, step, 0)

    return k


def kernel(x, fixed_weight, grad_weight):
    S, D = fixed_weight.shape
    G = grad_weight.shape[0]
    Bb, H = x.shape
    NC, NS = _sc_geometry()
    SA, GA = (S // 128) * 128, (G // 128) * 128

    fw_t = fixed_weight.T
    gw_t = grad_weight.T
    x_t = x.T.astype(jnp.int32)

    tail_rows = jnp.concatenate([fixed_weight[SA:], grad_weight[GA:]], axis=0)
    npad = (-tail_rows.shape[0]) % 4
    tail_rows = jnp.pad(tail_rows, ((0, npad), (0, 0)))
    tail_tab = tail_rows.reshape(-1, 4 * D)

    rmtab = _make_repack(S, G, D, SA, GA, NC, NS)(fw_t, gw_t)

    out_t = _make_lookup(Bb, H, D, S, G, SA, GA, NC, NS)(x_t, rmtab, tail_tab)
    return out_t.transpose(2, 0, 1)
